# baseline (speedup 1.0000x reference)
import functools

import jax
import jax.numpy as jnp
from jax import lax
from jax.experimental import pallas as pl
from jax.experimental.pallas import tpu as pltpu
from jax.experimental.pallas import tpu_sc as plsc


@functools.lru_cache(maxsize=None)
def _make_gather(V, D, B, chunk, nbuf):
    info = plsc.get_sparse_core_info()
    nc, ns = info.num_cores, info.num_subcores
    nw = nc * ns
    assert B % (8 * nw) == 0
    b_per_w = B // nw
    assert b_per_w % (chunk * nbuf) == 0
    n_chunks = b_per_w // chunk

    mesh = plsc.VectorSubcoreMesh(core_axis_name="c", subcore_axis_name="s")

    @functools.partial(
        pl.kernel,
        mesh=mesh,
        out_type=jax.ShapeDtypeStruct((B, D), jnp.float32),
        scratch_types=[
            pltpu.VMEM((nbuf, chunk), jnp.int32),
            pltpu.VMEM((nbuf, chunk, D), jnp.float32),
            pltpu.SemaphoreType.DMA((nbuf,)),
            pltpu.SemaphoreType.DMA((nbuf,)),
            pltpu.SemaphoreType.DMA((nbuf,)),
        ],
        compiler_params=pltpu.CompilerParams(use_tc_tiling_on_sc=False),
    )
    def gather_kernel(idx_hbm, table_hbm, out_hbm, idx_v, rows_v, isem, gsem, osem):
        wid = lax.axis_index("s") * nc + lax.axis_index("c")
        base = wid * b_per_w

        def idx_copy(c, b):
            off = pl.multiple_of(base + c * chunk, chunk)
            return pltpu.make_async_copy(
                idx_hbm.at[pl.ds(off, chunk)], idx_v.at[b], isem.at[b]
            )

        def out_copy(c, b):
            off = pl.multiple_of(base + c * chunk, chunk)
            return pltpu.make_async_copy(
                rows_v.at[b], out_hbm.at[pl.ds(off, chunk)], osem.at[b]
            )

        def gat_copy(b):
            return pltpu.make_async_copy(
                table_hbm.at[idx_v.at[b]], rows_v.at[b], gsem.at[b]
            )

        for b in range(nbuf):
            idx_copy(b, b).start()

        def body(i, carry):
            for b in range(nbuf):
                c = i * nbuf + b
                idx_copy(c, b).wait()

                @pl.when(c >= nbuf)
                def _():
                    out_copy(c, b).wait()

                gat_copy(b).start()

            for b in range(nbuf):
                c = i * nbuf + b
                gat_copy(b).wait()
                out_copy(c, b).start()

                @pl.when(c + nbuf < n_chunks)
                def _():
                    idx_copy(c + nbuf, b).start()

            return carry

        lax.fori_loop(0, n_chunks // nbuf, body, 0)

        for b in range(nbuf):
            out_copy(0, b).wait()

    return gather_kernel


def _tr_body(x_ref, o_ref):
    o_ref[0] = jnp.transpose(x_ref[0], (1, 0))


@functools.lru_cache(maxsize=None)
def _make_transpose(H, Bt, D, bb):
    assert Bt % bb == 0
    return pl.pallas_call(
        _tr_body,
        grid=(H, Bt // bb),
        in_specs=[pl.BlockSpec((1, bb, D), lambda h, j: (h, j, 0))],
        out_specs=pl.BlockSpec((1, D, bb), lambda h, j: (h, 0, j)),
        out_shape=jax.ShapeDtypeStruct((H, D, Bt), jnp.float32),
    )


def kernel(indices, table):
    batch, hist = indices.shape
    vocab, dim = table.shape
    n = batch * hist
    idx_t = jnp.transpose(indices, (1, 0)).reshape(n).astype(jnp.int32)
    rows = _make_gather(vocab, dim, n, 800, 4)(idx_t, table)
    rows3 = rows.reshape(hist, batch, dim)
    t = _make_transpose(hist, batch, dim, 2048)(rows3)
    return jnp.transpose(t, (2, 0, 1))


# speedup vs baseline: 1.0437x; 1.0437x over previous
import functools

import jax
import jax.numpy as jnp
from jax import lax
from jax.experimental import pallas as pl
from jax.experimental.pallas import tpu as pltpu
from jax.experimental.pallas import tpu_sc as plsc

_LANES = 16


@functools.lru_cache(maxsize=None)
def _make_gather(V, D, H, BT, chunk, nbuf):
    B = H * BT
    info = plsc.get_sparse_core_info()
    nc, ns = info.num_cores, info.num_subcores
    nw = nc * ns
    b_per_w = B // nw
    assert B % (8 * nw) == 0 and BT % chunk == 0 and b_per_w % (chunk * nbuf) == 0
    n_chunks = b_per_w // chunk
    k_per_h = BT // chunk
    nt = chunk // 128
    ns_t = D // 8
    tile_words = 8 * 128

    mesh = plsc.VectorSubcoreMesh(core_axis_name="c", subcore_axis_name="s")

    @functools.partial(
        pl.kernel,
        mesh=mesh,
        out_type=jax.ShapeDtypeStruct((H, ns_t, (BT // 128) * tile_words), jnp.float32),
        scratch_types=[
            pltpu.VMEM((nbuf, chunk), jnp.int32),
            pltpu.VMEM((nbuf, chunk, D), jnp.float32),
            pltpu.VMEM((nbuf, D * chunk), jnp.float32),
            pltpu.SemaphoreType.DMA((nbuf,)),
            pltpu.SemaphoreType.DMA((nbuf,)),
            pltpu.SemaphoreType.DMA((nbuf,)),
        ],
        compiler_params=pltpu.CompilerParams(
            use_tc_tiling_on_sc=False, needs_layout_passes=False
        ),
    )
    def gather_kernel(idx_hbm, table_hbm, out_hbm, idx_v, rows_v, tr_v, isem, gsem, osem):
        wid = lax.axis_index("s") * nc + lax.axis_index("c")
        base = wid * n_chunks
        lane = lax.broadcasted_iota(jnp.int32, (_LANES,), 0)

        def idx_copy(c, b):
            off = pl.multiple_of((base + c) * chunk, chunk)
            return pltpu.make_async_copy(
                idx_hbm.at[pl.ds(off, chunk)], idx_v.at[b], isem.at[b]
            )

        def gat_copy(b):
            return pltpu.make_async_copy(
                table_hbm.at[idx_v.at[b]], rows_v.at[b], gsem.at[b]
            )

        def out_copy(c, b, s, t):
            gci = base + c
            h = gci // k_per_h
            tbase = (gci % k_per_h) * nt
            src_off = pl.multiple_of((s * nt + t) * tile_words, tile_words)
            dst_off = pl.multiple_of((tbase + t) * tile_words, tile_words)
            return pltpu.make_async_copy(
                tr_v.at[b].at[pl.ds(src_off, tile_words)],
                out_hbm.at[h, s].at[pl.ds(dst_off, tile_words)],
                osem.at[b],
            )

        for b in range(nbuf):
            idx_copy(b, b).start()

        def transpose_chunk(b):
            rows = rows_v.at[b]
            tr = tr_v.at[b]

            def st_body(st, carry):
                s = st // nt
                t = lax.rem(st, nt)
                rbase = t * 128
                for r in range(8):
                    col = jnp.full((_LANES,), s * 8 + r, jnp.int32)
                    obase = (st * 8 + r) * 128
                    for cg in range(128 // _LANES):
                        ridx = lane + (rbase + cg * _LANES)
                        v = plsc.load_gather(rows, [ridx, col])
                        tr[pl.ds(pl.multiple_of(obase + cg * _LANES, _LANES), _LANES)] = v
                return carry

            lax.fori_loop(0, ns_t * nt, st_body, 0)

        def body(i, carry):
            for b in range(nbuf):
                c = i * nbuf + b
                idx_copy(c, b).wait()
                gat_copy(b).start()

            for b in range(nbuf):
                c = i * nbuf + b
                gat_copy(b).wait()

                @pl.when(c >= nbuf)
                def _():
                    for s in range(ns_t):
                        for t in range(nt):
                            out_copy(c, b, s, t).wait()

                transpose_chunk(b)
                for s in range(ns_t):
                    for t in range(nt):
                        out_copy(c, b, s, t).start()

                @pl.when(c + nbuf < n_chunks)
                def _():
                    idx_copy(c + nbuf, b).start()

            return carry

        lax.fori_loop(0, n_chunks // nbuf, body, 0)

        for b in range(nbuf):
            for s in range(ns_t):
                for t in range(nt):
                    out_copy(0, b, s, t).wait()

    return gather_kernel


def kernel(indices, table):
    batch, hist = indices.shape
    vocab, dim = table.shape
    n = batch * hist
    idx_t = jnp.transpose(indices, (1, 0)).reshape(n).astype(jnp.int32)
    t3 = _make_gather(vocab, dim, hist, batch, 512, 2)(idx_t, table)
    x = t3.reshape(hist, dim // 8, batch // 128, 8, 128)
    x = x.transpose(0, 1, 3, 2, 4)
    x = x.reshape(hist, dim, batch)
    return x.transpose(2, 0, 1)


# speedup vs baseline: 2.1348x; 2.0455x over previous
import functools

import jax
import jax.numpy as jnp
from jax import lax
from jax.experimental import pallas as pl
from jax.experimental.pallas import tpu as pltpu
from jax.experimental.pallas import tpu_sc as plsc

_LANES = 16


@functools.lru_cache(maxsize=None)
def _make_gather(V, D, H, BT, chunk, nbuf):
    B = H * BT
    info = plsc.get_sparse_core_info()
    nc, ns = info.num_cores, info.num_subcores
    nw = nc * ns
    b_per_w = B // nw
    assert B % (8 * nw) == 0 and BT % chunk == 0 and b_per_w % (chunk * nbuf) == 0
    n_chunks = b_per_w // chunk
    k_per_h = BT // chunk
    nt = chunk // 128
    ns_t = D // 8
    tile_words = 8 * 128

    mesh = plsc.VectorSubcoreMesh(core_axis_name="c", subcore_axis_name="s")

    @functools.partial(
        pl.kernel,
        mesh=mesh,
        out_type=jax.ShapeDtypeStruct((H, ns_t, BT // 128, 8, 128), jnp.float32),
        scratch_types=[
            pltpu.VMEM((nbuf, chunk), jnp.int32),
            pltpu.VMEM((nbuf, chunk, D), jnp.float32),
            pltpu.VMEM((nbuf, D, chunk), jnp.float32),
            pltpu.SemaphoreType.DMA((nbuf,)),
            pltpu.SemaphoreType.DMA((nbuf,)),
            pltpu.SemaphoreType.DMA((nbuf,)),
        ],
        compiler_params=pltpu.CompilerParams(
            use_tc_tiling_on_sc=False, needs_layout_passes=False
        ),
    )
    def gather_kernel(
        idx_hbm, table_hbm, out_hbm, idx_v, rows_v, tr_v, isem, gsem, osem
    ):
        wid = lax.axis_index("s") * nc + lax.axis_index("c")
        base = wid * n_chunks
        lane = lax.broadcasted_iota(jnp.int32, (_LANES,), 0)

        def idx_copy(c, b):
            off = pl.multiple_of((base + c) * chunk, chunk)
            return pltpu.make_async_copy(
                idx_hbm.at[pl.ds(off, chunk)], idx_v.at[b], isem.at[b]
            )

        def gat_copy(b):
            return pltpu.make_async_copy(
                table_hbm.at[idx_v.at[b]], rows_v.at[b], gsem.at[b]
            )

        def out_copy(c, b, s, t):
            gci = base + c
            h = gci // k_per_h
            tglob = (gci % k_per_h) * nt + t
            return pltpu.make_async_copy(
                tr_v.at[b].at[pl.ds(s * 8, 8), pl.ds(t * 128, 128)],
                out_hbm.at[h, s, tglob],
                osem.at[b],
            )

        for b in range(nbuf):
            idx_copy(b, b).start()

        def transpose_chunk(b):
            rows = rows_v.at[b]
            tr = tr_v.at[b]

            def b_body(bg, carry):
                bidx = lane + bg * _LANES
                for d0 in range(D):
                    dvec = lax.bitwise_and(lane + d0, D - 1)
                    v = plsc.load_gather(rows, [bidx, dvec])
                    plsc.store_scatter(tr, [dvec, bidx], v)
                return carry

            lax.fori_loop(0, chunk // _LANES, b_body, 0)

        def body(i, carry):
            for b in range(nbuf):
                c = i * nbuf + b
                idx_copy(c, b).wait()
                gat_copy(b).start()

            for b in range(nbuf):
                c = i * nbuf + b
                gat_copy(b).wait()

                @pl.when(c >= nbuf)
                def _():
                    for s in range(ns_t):
                        for t in range(nt):
                            out_copy(c, b, s, t).wait()

                transpose_chunk(b)
                for s in range(ns_t):
                    for t in range(nt):
                        out_copy(c, b, s, t).start()

                @pl.when(c + nbuf < n_chunks)
                def _():
                    idx_copy(c + nbuf, b).start()

            return carry

        lax.fori_loop(0, n_chunks // nbuf, body, 0)

        for b in range(nbuf):
            for s in range(ns_t):
                for t in range(nt):
                    out_copy(0, b, s, t).wait()

    return gather_kernel


def kernel(indices, table):
    batch, hist = indices.shape
    vocab, dim = table.shape
    n = batch * hist
    idx_t = jnp.transpose(indices, (1, 0)).reshape(n).astype(jnp.int32)
    t5 = _make_gather(vocab, dim, hist, batch, 512, 2)(idx_t, table)
    x = t5.transpose(0, 1, 3, 2, 4)
    x = x.reshape(hist, dim, batch)
    return x.transpose(2, 0, 1)


# speedup vs baseline: 2.9800x; 1.3959x over previous
import functools

import jax
import jax.numpy as jnp
from jax import lax
from jax.experimental import pallas as pl
from jax.experimental.pallas import tpu as pltpu
from jax.experimental.pallas import tpu_sc as plsc

_LANES = 16


@functools.lru_cache(maxsize=None)
def _make_gather(V, D, H, BT, chunk, nbuf):
    B = H * BT
    info = plsc.get_sparse_core_info()
    nc, ns = info.num_cores, info.num_subcores
    nw = nc * ns
    b_per_w = B // nw
    assert B % (8 * nw) == 0 and BT % chunk == 0 and b_per_w % (chunk * nbuf) == 0
    n_chunks = b_per_w // chunk
    k_per_h = BT // chunk
    nt = chunk // 128
    ns_t = D // 8
    tile_words = 8 * 128

    mesh = plsc.VectorSubcoreMesh(core_axis_name="c", subcore_axis_name="s")

    @functools.partial(
        pl.kernel,
        mesh=mesh,
        out_type=jax.ShapeDtypeStruct((H, ns_t, BT // 128, 8, 128), jnp.float32),
        scratch_types=[
            pltpu.VMEM((nbuf, chunk), jnp.int32),
            pltpu.VMEM((nbuf, chunk, D), jnp.float32),
            pltpu.VMEM((nbuf, D, chunk), jnp.float32),
            pltpu.SemaphoreType.DMA((nbuf,)),
            pltpu.SemaphoreType.DMA((nbuf,)),
            pltpu.SemaphoreType.DMA((nbuf,)),
        ],
        compiler_params=pltpu.CompilerParams(
            use_tc_tiling_on_sc=False, needs_layout_passes=False
        ),
    )
    def gather_kernel(
        idx_hbm, table_hbm, out_hbm, idx_v, rows_v, tr_v, isem, gsem, osem
    ):
        wid = lax.axis_index("s") * nc + lax.axis_index("c")
        base = wid * n_chunks
        lane = lax.broadcasted_iota(jnp.int32, (_LANES,), 0)

        def idx_copy(c, b):
            off = pl.multiple_of((base + c) * chunk, chunk)
            return pltpu.make_async_copy(
                idx_hbm.at[pl.ds(off, chunk)], idx_v.at[b], isem.at[b]
            )

        def gat_copy(b):
            return pltpu.make_async_copy(
                table_hbm.at[idx_v.at[b]], rows_v.at[b], gsem.at[b]
            )

        def out_copy(c, b, s, t):
            gci = base + c
            h = gci // k_per_h
            tglob = (gci % k_per_h) * nt + t
            return pltpu.make_async_copy(
                tr_v.at[b].at[pl.ds(s * 8, 8), pl.ds(t * 128, 128)],
                out_hbm.at[h, s, tglob],
                osem.at[b],
            )

        for b in range(nbuf):
            idx_copy(b, b).start()

        def transpose_chunk(b):
            rows = rows_v.at[b]
            tr = tr_v.at[b]

            def b_body(bg, carry):
                bidx = lane + bg * _LANES
                for d0 in range(0, D, 4):
                    dvs = [lax.bitwise_and(lane + d0 + k, D - 1) for k in range(4)]
                    vs = [plsc.load_gather(rows, [bidx, dv]) for dv in dvs]
                    for dv, v in zip(dvs, vs):
                        plsc.store_scatter(tr, [dv, bidx], v)
                return carry

            lax.fori_loop(0, chunk // _LANES, b_body, 0)

        def body(i, carry):
            for b in range(nbuf):
                c = i * nbuf + b
                idx_copy(c, b).wait()
                gat_copy(b).start()

            for b in range(nbuf):
                c = i * nbuf + b
                gat_copy(b).wait()

                @pl.when(c >= nbuf)
                def _():
                    for s in range(ns_t):
                        for t in range(nt):
                            out_copy(c, b, s, t).wait()

                transpose_chunk(b)
                for s in range(ns_t):
                    for t in range(nt):
                        out_copy(c, b, s, t).start()

                @pl.when(c + nbuf < n_chunks)
                def _():
                    idx_copy(c + nbuf, b).start()

            return carry

        lax.fori_loop(0, n_chunks // nbuf, body, 0)

        for b in range(nbuf):
            for s in range(ns_t):
                for t in range(nt):
                    out_copy(0, b, s, t).wait()

    return gather_kernel


def kernel(indices, table):
    batch, hist = indices.shape
    vocab, dim = table.shape
    n = batch * hist
    idx_t = jnp.transpose(indices, (1, 0)).reshape(n).astype(jnp.int32)
    t5 = _make_gather(vocab, dim, hist, batch, 512, 2)(idx_t, table)
    x = t5.transpose(0, 1, 3, 2, 4)
    x = x.reshape(hist, dim, batch)
    return x.transpose(2, 0, 1)


# speedup vs baseline: 2.9864x; 1.0022x over previous
import functools

import jax
import jax.numpy as jnp
from jax import lax
from jax.experimental import pallas as pl
from jax.experimental.pallas import tpu as pltpu
from jax.experimental.pallas import tpu_sc as plsc

_LANES = 16


@functools.lru_cache(maxsize=None)
def _make_gather(V, D, H, BT, chunk, nbuf):
    B = H * BT
    info = plsc.get_sparse_core_info()
    nc, ns = info.num_cores, info.num_subcores
    nw = nc * ns
    b_per_w = B // nw
    assert B % (8 * nw) == 0 and BT % chunk == 0 and b_per_w % (chunk * nbuf) == 0
    n_chunks = b_per_w // chunk
    k_per_h = BT // chunk
    nt = chunk // 128
    ns_t = D // 8
    tile_words = 8 * 128

    mesh = plsc.VectorSubcoreMesh(core_axis_name="c", subcore_axis_name="s")

    @functools.partial(
        pl.kernel,
        mesh=mesh,
        out_type=jax.ShapeDtypeStruct((H, ns_t, BT // 128, 8, 128), jnp.float32),
        scratch_types=[
            pltpu.VMEM((nbuf, chunk), jnp.int32),
            pltpu.VMEM((nbuf, chunk, D), jnp.float32),
            pltpu.VMEM((nbuf, D, chunk), jnp.float32),
            pltpu.SemaphoreType.DMA((nbuf,)),
            pltpu.SemaphoreType.DMA((nbuf,)),
            pltpu.SemaphoreType.DMA((nbuf,)),
        ],
        compiler_params=pltpu.CompilerParams(
            use_tc_tiling_on_sc=False, needs_layout_passes=False
        ),
    )
    def gather_kernel(
        idx_hbm, table_hbm, out_hbm, idx_v, rows_v, tr_v, isem, gsem, osem
    ):
        wid = lax.axis_index("s") * nc + lax.axis_index("c")
        base = wid * n_chunks
        lane = lax.broadcasted_iota(jnp.int32, (_LANES,), 0)

        def idx_copy(c, b):
            off = pl.multiple_of((base + c) * chunk, chunk)
            return pltpu.make_async_copy(
                idx_hbm.at[pl.ds(off, chunk)], idx_v.at[b], isem.at[b]
            )

        def gat_copy(b):
            return pltpu.make_async_copy(
                table_hbm.at[idx_v.at[b]], rows_v.at[b], gsem.at[b]
            )

        def out_copy(c, b, s, t):
            gci = base + c
            h = gci // k_per_h
            tglob = (gci % k_per_h) * nt + t
            return pltpu.make_async_copy(
                tr_v.at[b].at[pl.ds(s * 8, 8), pl.ds(t * 128, 128)],
                out_hbm.at[h, s, tglob],
                osem.at[b],
            )

        for b in range(nbuf):
            idx_copy(b, b).start()

        def transpose_chunk(b):
            rows = rows_v.at[b]
            tr = tr_v.at[b]

            def b_body(bg, carry):
                bidx = lane + bg * _LANES
                for d0 in range(0, D, 8):
                    dvs = [lax.bitwise_and(lane + d0 + k, D - 1) for k in range(8)]
                    vs = [plsc.load_gather(rows, [bidx, dv]) for dv in dvs]
                    for dv, v in zip(dvs, vs):
                        plsc.store_scatter(tr, [dv, bidx], v)
                return carry

            lax.fori_loop(0, chunk // _LANES, b_body, 0)

        def body(i, carry):
            for b in range(nbuf):
                c = i * nbuf + b
                idx_copy(c, b).wait()
                gat_copy(b).start()

            for b in range(nbuf):
                c = i * nbuf + b
                gat_copy(b).wait()

                @pl.when(c >= nbuf)
                def _():
                    for s in range(ns_t):
                        for t in range(nt):
                            out_copy(c, b, s, t).wait()

                transpose_chunk(b)
                for s in range(ns_t):
                    for t in range(nt):
                        out_copy(c, b, s, t).start()

                @pl.when(c + nbuf < n_chunks)
                def _():
                    idx_copy(c + nbuf, b).start()

            return carry

        lax.fori_loop(0, n_chunks // nbuf, body, 0)

        for b in range(nbuf):
            for s in range(ns_t):
                for t in range(nt):
                    out_copy(0, b, s, t).wait()

    return gather_kernel


def kernel(indices, table):
    batch, hist = indices.shape
    vocab, dim = table.shape
    n = batch * hist
    idx_t = jnp.transpose(indices, (1, 0)).reshape(n).astype(jnp.int32)
    t5 = _make_gather(vocab, dim, hist, batch, 512, 2)(idx_t, table)
    x = t5.transpose(0, 1, 3, 2, 4)
    x = x.reshape(hist, dim, batch)
    return x.transpose(2, 0, 1)
